# baseline (speedup 1.0000x reference)
import functools

import jax
import jax.numpy as jnp
from jax import lax
from jax.experimental import pallas as pl
from jax.experimental.pallas import tpu as pltpu
from jax.experimental.pallas import tpu_sc as plsc

_NU = 2048
_NI = 2048
_R = 5
_B = 1024
_H = 64
_SD = 128
_IN = 64
_TW = _R * _H + _SD

_NC = 2
_NS = 16
_NW = _NC * _NS
_RPW = _B // _NW
_ACH = 8
_SLOT = _R * _ACH
_SLOTP = 48
_NT = _RPW // _ACH


def _sc_rows(adj2d, u):
    mesh = plsc.VectorSubcoreMesh(core_axis_name="c", subcore_axis_name="s")
    out_type = jax.ShapeDtypeStruct((_B, _NI), jnp.float32)
    scratch = [
        pltpu.VMEM((_RPW + 16,), jnp.int32),
        pltpu.VMEM((32,), jnp.int32),
        pltpu.VMEM((_SLOTP,), jnp.int32),
        pltpu.VMEM((_SLOT, _NI), jnp.float32),
        pltpu.VMEM((_ACH, _NI), jnp.float32),
        pltpu.SemaphoreType.DMA,
    ]

    @functools.partial(pl.kernel, out_type=out_type, mesh=mesh,
                       scratch_types=scratch)
    def k(adj_h, u_h, g_h, u_idx, pbuf, aidx, arows, grows, sem):
        wid = lax.axis_index("s") * _NC + lax.axis_index("c")
        base = wid * _RPW
        pltpu.sync_copy(u_h.at[pl.ds(base, _RPW)], u_idx.at[pl.ds(0, _RPW)])

        for t in range(_NT):
            x16 = u_idx[pl.ds(t * _ACH, 16)]
            pbuf[pl.ds(0, 16)] = x16
            pbuf[pl.ds(8, 16)] = x16
            w = pbuf[pl.ds(0, 16)]
            for h in range(_SLOTP // 16):
                kvec = lax.iota(jnp.int32, 16) + h * 16
                c = jnp.minimum(lax.shift_right_logical(kvec, 3), _R - 1)
                aidx[pl.ds(h * 16, 16)] = w + c * _NU
            pltpu.async_copy(
                adj_h.at[aidx.at[pl.ds(0, _SLOT)]], arows, sem).wait()

            def jblock(jb, _):
                for r in range(_ACH):
                    acc = arows[r, pl.ds(jb * 16, 16)]
                    for c in range(1, _R):
                        acc = acc + float(c + 1) * arows[
                            c * _ACH + r, pl.ds(jb * 16, 16)]
                    grows[r, pl.ds(jb * 16, 16)] = acc
                return ()

            lax.fori_loop(0, _NI // 16, jblock, ())
            pltpu.sync_copy(grows, g_h.at[pl.ds(base + t * _ACH, _ACH)])

    return k(adj2d, u)


def _lookup_tc(u2, v2, utab, vtab):
    def body(u_ref, v_ref, ut_ref, vt_ref, ux_ref, vx_ref):
        ohu = (u_ref[...] == lax.broadcasted_iota(jnp.int32, (_B, _NU), 1)
               ).astype(jnp.float32)
        ohv = (v_ref[...] == lax.broadcasted_iota(jnp.int32, (_B, _NI), 1)
               ).astype(jnp.float32)
        ux_ref[...] = jnp.dot(ohu, ut_ref[...],
                              preferred_element_type=jnp.float32)
        vx_ref[...] = jnp.dot(ohv, vt_ref[...],
                              preferred_element_type=jnp.float32)

    return pl.pallas_call(
        body,
        out_shape=(
            jax.ShapeDtypeStruct((_B, _TW), jnp.float32),
            jax.ShapeDtypeStruct((_B, _TW), jnp.float32),
        ),
    )(u2, v2, utab, vtab)


def _encoder(g, v2, uxs, vxs, Wu1, bu1, Wv1, bv1, Wu2, Wv2):
    def body(g_ref, v_ref, uxs_ref, vxs_ref, wu1_ref, bu1_ref,
             wv1_ref, bv1_ref, wu2_ref, wv2_ref, mcls_ref, uh_ref, vh_ref):
        ohv = (v_ref[...] == lax.broadcasted_iota(jnp.int32, (_B, _NI), 1)
               ).astype(jnp.bfloat16)
        m = lax.dot_general(
            g_ref[...].astype(jnp.bfloat16), ohv, (((1,), (1,)), ((), ())),
            preferred_element_type=jnp.float32)
        mcls_ref[...] = m
        mask = (m > 0.0).astype(jnp.float32)
        du = jnp.sum(mask, axis=1)
        di = jnp.sum(mask, axis=0)
        cu = jnp.where(du > 0.0, 1.0 / du, 0.0)
        cv = jnp.where(di > 0.0, 1.0 / di, 0.0)
        accu = jnp.zeros((_B, _H), jnp.float32)
        accv = jnp.zeros((_B, _H), jnp.float32)
        for c in range(_R):
            mc = (m == float(c + 1)).astype(jnp.float32)
            accu = accu + jnp.dot(mc, vxs_ref[:, c * _H:(c + 1) * _H],
                                  preferred_element_type=jnp.float32)
            accv = accv + lax.dot_general(
                mc, uxs_ref[:, c * _H:(c + 1) * _H],
                (((0,), (0,)), ((), ())),
                preferred_element_type=jnp.float32)
        zu = jnp.maximum(accu * cu[:, None], 0.0)
        zv = jnp.maximum(accv * cv[:, None], 0.0)
        ufm = jnp.maximum(jnp.dot(uxs_ref[:, _R * _H:], wu1_ref[...],
                                  preferred_element_type=jnp.float32)
                          + bu1_ref[...], 0.0)
        vfm = jnp.maximum(jnp.dot(vxs_ref[:, _R * _H:], wv1_ref[...],
                                  preferred_element_type=jnp.float32)
                          + bv1_ref[...], 0.0)
        uh_ref[...] = jnp.maximum(
            jnp.dot(zu, wu2_ref[:_H], preferred_element_type=jnp.float32)
            + jnp.dot(ufm, wu2_ref[_H:], preferred_element_type=jnp.float32),
            0.0)
        vh_ref[...] = jnp.maximum(
            jnp.dot(zv, wv2_ref[:_H], preferred_element_type=jnp.float32)
            + jnp.dot(vfm, wv2_ref[_H:], preferred_element_type=jnp.float32),
            0.0)

    return pl.pallas_call(
        body,
        out_shape=(
            jax.ShapeDtypeStruct((_B, _B), jnp.float32),
            jax.ShapeDtypeStruct((_B, _H), jnp.float32),
            jax.ShapeDtypeStruct((_B, _H), jnp.float32),
        ),
    )(g, v2, uxs, vxs, Wu1, bu1, Wv1, bv1, Wu2, Wv2)


_BJ = 256


def _decoder(uh, vh, mcls, P, a):
    def body(uh_ref, vh_ref, mcls_ref, p_ref, a_ref,
             mh_ref, sl_ref, sm_ref, sq_ref):
        j = pl.program_id(0)
        uh_ = uh_ref[...]
        vh_ = vh_ref[...]
        m = mcls_ref[...]
        p0 = p_ref[0]
        p1 = p_ref[1]
        logits = []
        for c in range(_R):
            q = a_ref[c, 0] * p0 + a_ref[c, 1] * p1
            t = jnp.dot(uh_, q, preferred_element_type=jnp.float32)
            logits.append(lax.dot_general(
                t, vh_, (((1,), (1,)), ((), ())),
                preferred_element_type=jnp.float32))
        mx = logits[0]
        for c in range(1, _R):
            mx = jnp.maximum(mx, logits[c])
        es = [jnp.exp(l - mx) for l in logits]
        s = es[0]
        for c in range(1, _R):
            s = s + es[c]
        num = es[0]
        for c in range(1, _R):
            num = num + float(c + 1) * es[c]
        mh = num / s
        mh_ref[...] = mh
        logs = jnp.log(s)
        lossn = jnp.zeros((), jnp.float32)
        for c in range(_R):
            lossn = lossn + jnp.sum(
                jnp.where(m == float(c + 1), logits[c] - mx - logs, 0.0))
        mask = (m > 0.0).astype(jnp.float32)
        msum = jnp.sum(mask)
        sq = jnp.sum(((mh - m) * mask) ** 2)

        @pl.when(j == 0)
        def _():
            sl_ref[0, 0] = 0.0
            sm_ref[0, 0] = 0.0
            sq_ref[0, 0] = 0.0

        sl_ref[0, 0] += lossn
        sm_ref[0, 0] += msum
        sq_ref[0, 0] += sq

    return pl.pallas_call(
        body,
        grid=(_B // _BJ,),
        in_specs=[
            pl.BlockSpec((_B, _H), lambda j: (0, 0)),
            pl.BlockSpec((_BJ, _H), lambda j: (j, 0)),
            pl.BlockSpec((_B, _BJ), lambda j: (0, j)),
            pl.BlockSpec((2, _H, _H), lambda j: (0, 0, 0)),
            pl.BlockSpec(memory_space=pltpu.SMEM),
        ],
        out_specs=(
            pl.BlockSpec((_B, _BJ), lambda j: (0, j)),
            pl.BlockSpec(memory_space=pltpu.SMEM),
            pl.BlockSpec(memory_space=pltpu.SMEM),
            pl.BlockSpec(memory_space=pltpu.SMEM),
        ),
        out_shape=(
            jax.ShapeDtypeStruct((_B, _B), jnp.float32),
            jax.ShapeDtypeStruct((1, 1), jnp.float32),
            jax.ShapeDtypeStruct((1, 1), jnp.float32),
            jax.ShapeDtypeStruct((1, 1), jnp.float32),
        ),
    )(uh, vh, mcls, P, a)


def kernel(u, v, r, u_features, v_features, adj_train, Wu, Wv, Wu1, bu1,
           Wv1, bv1, Wu2, Wv2, P, a):
    del r
    u32 = u.astype(jnp.int32)
    v32 = v.astype(jnp.int32)
    adj2d = adj_train.reshape(_R * _NU, _NI)

    utab = jnp.concatenate(
        [Wu.transpose(1, 0, 2).reshape(_NU, _R * _H), u_features], axis=1)
    vtab = jnp.concatenate(
        [Wv.transpose(1, 0, 2).reshape(_NI, _R * _H), v_features], axis=1)

    g = _sc_rows(adj2d, u32)
    uxs, vxs = _lookup_tc(u32.reshape(_B, 1), v32.reshape(_B, 1), utab, vtab)
    mcls, uh, vh = _encoder(g, v32.reshape(_B, 1), uxs, vxs,
                            Wu1, bu1.reshape(1, _IN), Wv1, bv1.reshape(1, _IN),
                            Wu2, Wv2)
    m_hat, sl, sm, sq = _decoder(uh, vh, mcls, P, a)

    denom = jnp.maximum(sm[0, 0], 1.0)
    loss = -sl[0, 0] / denom
    rmse = jnp.sqrt(sq[0, 0] / denom)
    return (m_hat, loss, rmse)


# speedup vs baseline: 1.1308x; 1.1308x over previous
import functools

import jax
import jax.numpy as jnp
from jax import lax
from jax.experimental import pallas as pl
from jax.experimental.pallas import tpu as pltpu
from jax.experimental.pallas import tpu_sc as plsc

_NU = 2048
_NI = 2048
_R = 5
_B = 1024
_H = 64
_SD = 128
_IN = 64
_TW = _R * _H + _SD

_NC = 2
_NS = 16
_NW = _NC * _NS
_RPW = _B // _NW
_ACH = 8
_SLOT = _R * _ACH
_SLOTP = 48
_NT = _RPW // _ACH
_NIH = _NI // 2


def _sc_rows(adj2d, u):
    mesh = plsc.VectorSubcoreMesh(core_axis_name="c", subcore_axis_name="s")
    out_type = jax.ShapeDtypeStruct((_B, _NI), jnp.float32)
    scratch = [
        pltpu.VMEM((_RPW + 16,), jnp.int32),
        pltpu.VMEM((32,), jnp.int32),
        pltpu.VMEM((_SLOTP,), jnp.int32),
        pltpu.VMEM((_SLOTP,), jnp.int32),
        pltpu.VMEM((_SLOT, _NIH), jnp.float32),
        pltpu.VMEM((_SLOT, _NIH), jnp.float32),
        pltpu.VMEM((_ACH, _NI), jnp.float32),
        pltpu.SemaphoreType.DMA,
        pltpu.SemaphoreType.DMA,
    ]

    @functools.partial(pl.kernel, out_type=out_type, mesh=mesh,
                       scratch_types=scratch)
    def k(adj_h, u_h, g_h, u_idx, pbuf, aidx0, aidx1, arows0, arows1, grows,
          sem0, sem1):
        wid = lax.axis_index("s") * _NC + lax.axis_index("c")
        base = wid * _RPW
        pltpu.sync_copy(u_h.at[pl.ds(base, _RPW)], u_idx.at[pl.ds(0, _RPW)])
        aidx = (aidx0, aidx1)
        arows = (arows0, arows1)
        sems = (sem0, sem1)

        def build(t):
            a = aidx[t % 2]
            x16 = u_idx[pl.ds(t * _ACH, 16)]
            pbuf[pl.ds(0, 16)] = x16
            pbuf[pl.ds(8, 16)] = x16
            w = pbuf[pl.ds(0, 16)]
            for hh in range(_SLOTP // 16):
                kvec = lax.iota(jnp.int32, 16) + hh * 16
                c = jnp.minimum(lax.shift_right_logical(kvec, 3), _R - 1)
                a[pl.ds(hh * 16, 16)] = w + c * _NU

        def issue(t, h):
            return pltpu.async_copy(
                adj_h.at[aidx[t % 2].at[pl.ds(0, _SLOT)],
                         pl.ds(h * _NIH, _NIH)],
                arows[h], sems[h])

        seq = [(t, h) for t in range(_NT) for h in range(2)]
        build(0)
        cps = {(0, 0): issue(0, 0)}
        for i, (t, h) in enumerate(seq):
            if i + 1 < len(seq):
                tn, hn = seq[i + 1]
                if hn == 0:
                    build(tn)
                cps[(tn, hn)] = issue(tn, hn)
            cps[(t, h)].wait()
            rows = arows[h]

            def jblock(jb, _):
                for r in range(_ACH):
                    acc = rows[r, pl.ds(jb * 16, 16)]
                    for c in range(1, _R):
                        acc = acc + float(c + 1) * rows[
                            c * _ACH + r, pl.ds(jb * 16, 16)]
                    grows[r, pl.ds(h * _NIH + jb * 16, 16)] = acc
                return ()

            lax.fori_loop(0, _NIH // 16, jblock, ())
            if h == 1:
                pltpu.sync_copy(grows, g_h.at[pl.ds(base + t * _ACH, _ACH)])

    return k(adj2d, u)


def _lookup_tc(u2, v2, utab, vtab):
    def body(u_ref, v_ref, ut_ref, vt_ref, ux_ref, vx_ref):
        ohu = (u_ref[...] == lax.broadcasted_iota(jnp.int32, (_B, _NU), 1)
               ).astype(jnp.float32)
        ohv = (v_ref[...] == lax.broadcasted_iota(jnp.int32, (_B, _NI), 1)
               ).astype(jnp.float32)
        ux_ref[...] = jnp.dot(ohu, ut_ref[...],
                              preferred_element_type=jnp.float32)
        vx_ref[...] = jnp.dot(ohv, vt_ref[...],
                              preferred_element_type=jnp.float32)

    return pl.pallas_call(
        body,
        out_shape=(
            jax.ShapeDtypeStruct((_B, _TW), jnp.float32),
            jax.ShapeDtypeStruct((_B, _TW), jnp.float32),
        ),
    )(u2, v2, utab, vtab)


def _encoder(g, v2, uxs, vxs, Wu1, bu1, Wv1, bv1, Wu2, Wv2):
    def body(g_ref, v_ref, uxs_ref, vxs_ref, wu1_ref, bu1_ref,
             wv1_ref, bv1_ref, wu2_ref, wv2_ref, mcls_ref, uh_ref, vh_ref):
        ohv = (v_ref[...] == lax.broadcasted_iota(jnp.int32, (_B, _NI), 1)
               ).astype(jnp.bfloat16)
        m = lax.dot_general(
            g_ref[...].astype(jnp.bfloat16), ohv, (((1,), (1,)), ((), ())),
            preferred_element_type=jnp.float32)
        mcls_ref[...] = m
        mask = (m > 0.0).astype(jnp.float32)
        du = jnp.sum(mask, axis=1)
        di = jnp.sum(mask, axis=0)
        cu = jnp.where(du > 0.0, 1.0 / du, 0.0)
        cv = jnp.where(di > 0.0, 1.0 / di, 0.0)
        accu = jnp.zeros((_B, _H), jnp.float32)
        accv = jnp.zeros((_B, _H), jnp.float32)
        for c in range(_R):
            mc = (m == float(c + 1)).astype(jnp.float32)
            accu = accu + jnp.dot(mc, vxs_ref[:, c * _H:(c + 1) * _H],
                                  preferred_element_type=jnp.float32)
            accv = accv + lax.dot_general(
                mc, uxs_ref[:, c * _H:(c + 1) * _H],
                (((0,), (0,)), ((), ())),
                preferred_element_type=jnp.float32)
        zu = jnp.maximum(accu * cu[:, None], 0.0)
        zv = jnp.maximum(accv * cv[:, None], 0.0)
        ufm = jnp.maximum(jnp.dot(uxs_ref[:, _R * _H:], wu1_ref[...],
                                  preferred_element_type=jnp.float32)
                          + bu1_ref[...], 0.0)
        vfm = jnp.maximum(jnp.dot(vxs_ref[:, _R * _H:], wv1_ref[...],
                                  preferred_element_type=jnp.float32)
                          + bv1_ref[...], 0.0)
        uh_ref[...] = jnp.maximum(
            jnp.dot(zu, wu2_ref[:_H], preferred_element_type=jnp.float32)
            + jnp.dot(ufm, wu2_ref[_H:], preferred_element_type=jnp.float32),
            0.0)
        vh_ref[...] = jnp.maximum(
            jnp.dot(zv, wv2_ref[:_H], preferred_element_type=jnp.float32)
            + jnp.dot(vfm, wv2_ref[_H:], preferred_element_type=jnp.float32),
            0.0)

    return pl.pallas_call(
        body,
        out_shape=(
            jax.ShapeDtypeStruct((_B, _B), jnp.float32),
            jax.ShapeDtypeStruct((_B, _H), jnp.float32),
            jax.ShapeDtypeStruct((_B, _H), jnp.float32),
        ),
    )(g, v2, uxs, vxs, Wu1, bu1, Wv1, bv1, Wu2, Wv2)


_BJ = 256


def _decoder(uh, vh, mcls, P, a):
    def body(uh_ref, vh_ref, mcls_ref, p_ref, a_ref,
             mh_ref, sl_ref, sm_ref, sq_ref):
        j = pl.program_id(0)
        uh_ = uh_ref[...]
        vh_ = vh_ref[...]
        m = mcls_ref[...]
        p0 = p_ref[0]
        p1 = p_ref[1]
        logits = []
        for c in range(_R):
            q = a_ref[c, 0] * p0 + a_ref[c, 1] * p1
            t = jnp.dot(uh_, q, preferred_element_type=jnp.float32)
            logits.append(lax.dot_general(
                t, vh_, (((1,), (1,)), ((), ())),
                preferred_element_type=jnp.float32))
        mx = logits[0]
        for c in range(1, _R):
            mx = jnp.maximum(mx, logits[c])
        es = [jnp.exp(l - mx) for l in logits]
        s = es[0]
        for c in range(1, _R):
            s = s + es[c]
        num = es[0]
        for c in range(1, _R):
            num = num + float(c + 1) * es[c]
        mh = num / s
        mh_ref[...] = mh
        logs = jnp.log(s)
        lossn = jnp.zeros((), jnp.float32)
        for c in range(_R):
            lossn = lossn + jnp.sum(
                jnp.where(m == float(c + 1), logits[c] - mx - logs, 0.0))
        mask = (m > 0.0).astype(jnp.float32)
        msum = jnp.sum(mask)
        sq = jnp.sum(((mh - m) * mask) ** 2)

        @pl.when(j == 0)
        def _():
            sl_ref[0, 0] = 0.0
            sm_ref[0, 0] = 0.0
            sq_ref[0, 0] = 0.0

        sl_ref[0, 0] += lossn
        sm_ref[0, 0] += msum
        sq_ref[0, 0] += sq

    return pl.pallas_call(
        body,
        grid=(_B // _BJ,),
        in_specs=[
            pl.BlockSpec((_B, _H), lambda j: (0, 0)),
            pl.BlockSpec((_BJ, _H), lambda j: (j, 0)),
            pl.BlockSpec((_B, _BJ), lambda j: (0, j)),
            pl.BlockSpec((2, _H, _H), lambda j: (0, 0, 0)),
            pl.BlockSpec(memory_space=pltpu.SMEM),
        ],
        out_specs=(
            pl.BlockSpec((_B, _BJ), lambda j: (0, j)),
            pl.BlockSpec(memory_space=pltpu.SMEM),
            pl.BlockSpec(memory_space=pltpu.SMEM),
            pl.BlockSpec(memory_space=pltpu.SMEM),
        ),
        out_shape=(
            jax.ShapeDtypeStruct((_B, _B), jnp.float32),
            jax.ShapeDtypeStruct((1, 1), jnp.float32),
            jax.ShapeDtypeStruct((1, 1), jnp.float32),
            jax.ShapeDtypeStruct((1, 1), jnp.float32),
        ),
    )(uh, vh, mcls, P, a)


def kernel(u, v, r, u_features, v_features, adj_train, Wu, Wv, Wu1, bu1,
           Wv1, bv1, Wu2, Wv2, P, a):
    del r
    u32 = u.astype(jnp.int32)
    v32 = v.astype(jnp.int32)
    adj2d = adj_train.reshape(_R * _NU, _NI)

    utab = jnp.concatenate(
        [Wu.transpose(1, 0, 2).reshape(_NU, _R * _H), u_features], axis=1)
    vtab = jnp.concatenate(
        [Wv.transpose(1, 0, 2).reshape(_NI, _R * _H), v_features], axis=1)

    g = _sc_rows(adj2d, u32)
    uxs, vxs = _lookup_tc(u32.reshape(_B, 1), v32.reshape(_B, 1), utab, vtab)
    mcls, uh, vh = _encoder(g, v32.reshape(_B, 1), uxs, vxs,
                            Wu1, bu1.reshape(1, _IN), Wv1, bv1.reshape(1, _IN),
                            Wu2, Wv2)
    m_hat, sl, sm, sq = _decoder(uh, vh, mcls, P, a)

    denom = jnp.maximum(sm[0, 0], 1.0)
    loss = -sl[0, 0] / denom
    rmse = jnp.sqrt(sq[0, 0] / denom)
    return (m_hat, loss, rmse)
